# Initial kernel scaffold; baseline (speedup 1.0000x reference)
import jax
import jax.numpy as jnp
from jax.experimental import pallas as pl


def kernel(spd_dense, degree_dense, spd_emb, degree_emb):
    raise NotImplementedError("write your pallas kernel here")


# speedup vs baseline: 119.0140x; 119.0140x over previous
import functools

import jax
import jax.numpy as jnp
from jax import lax
from jax.experimental import pallas as pl
from jax.experimental.pallas import tpu as pltpu
from jax.experimental.pallas import tpu_sc as plsc

_SPD_MAX = 4
_IDX_CHUNK = 128


def _deg_gather_sc(table, idx_flat):
    bn = idx_flat.shape[0]
    h = table.shape[1]
    info = plsc.get_sparse_core_info()
    nw = info.num_cores * info.num_subcores
    per = bn // nw
    nch = per // _IDX_CHUNK
    mesh = plsc.VectorSubcoreMesh(core_axis_name="c", subcore_axis_name="s")

    @functools.partial(
        pl.kernel,
        mesh=mesh,
        compiler_params=pltpu.CompilerParams(use_tc_tiling_on_sc=False),
        out_type=jax.ShapeDtypeStruct((bn, h), jnp.float32),
        scratch_types=[
            pltpu.VMEM((per,), jnp.int32),
            pltpu.VMEM((per, h), jnp.float32),
            pltpu.SemaphoreType.DMA,
        ],
    )
    def k(table_hbm, idx_hbm, out_hbm, idx_v, rows_v, sem):
        wid = lax.axis_index("s") * info.num_cores + lax.axis_index("c")
        base = wid * per
        pltpu.sync_copy(idx_hbm.at[pl.ds(base, per)], idx_v)
        copies = [
            pltpu.async_copy(
                table_hbm.at[idx_v.at[pl.ds(j * _IDX_CHUNK, _IDX_CHUNK)]],
                rows_v.at[pl.ds(j * _IDX_CHUNK, _IDX_CHUNK)],
                sem,
            )
            for j in range(nch)
        ]
        for cp in copies:
            cp.wait()
        pltpu.sync_copy(rows_v, out_hbm.at[pl.ds(base, per)])

    return k(table, idx_flat)


def _bias_body(emb_ref, spd_ref, deg_ref, degt_ref, out_ref):
    s = jnp.minimum(spd_ref[0], _SPD_MAX)
    masks = [(s == kk).astype(jnp.float32) for kk in range(1, _SPD_MAX + 1)]
    deg = deg_ref[0]
    degt = degt_ref[0]
    n_heads = degt.shape[0]
    for hh in range(n_heads):
        t0 = emb_ref[0, hh]
        acc = deg[:, hh : hh + 1] + degt[hh : hh + 1, :] + t0
        for kk in range(1, _SPD_MAX + 1):
            acc = acc + masks[kk - 1] * (emb_ref[kk, hh] - t0)
        out_ref[0, hh] = acc


def _materialize_bias(spd_dense, spd_emb, deg, degt, *, interpret=False):
    b, n, _ = spd_dense.shape
    h = spd_emb.shape[1]
    ti = 256
    grid = (b, n // ti)
    return pl.pallas_call(
        _bias_body,
        grid=grid,
        in_specs=[
            pl.BlockSpec(memory_space=pltpu.SMEM),
            pl.BlockSpec((1, ti, n), lambda bb, ii: (bb, ii, 0)),
            pl.BlockSpec((1, ti, h), lambda bb, ii: (bb, ii, 0)),
            pl.BlockSpec((1, h, n), lambda bb, ii: (bb, 0, 0)),
        ],
        out_specs=pl.BlockSpec((1, h, ti, n), lambda bb, ii: (bb, 0, ii, 0)),
        out_shape=jax.ShapeDtypeStruct((b, h, n, n), jnp.float32),
        compiler_params=pltpu.CompilerParams(
            dimension_semantics=("parallel", "parallel"),
        ),
        interpret=interpret,
    )(spd_emb, spd_dense, deg, degt)


def kernel(spd_dense, degree_dense, spd_emb, degree_emb):
    b, n, _ = spd_dense.shape
    h = spd_emb.shape[1]
    idx = degree_dense.astype(jnp.int32).reshape(b * n)
    deg = _deg_gather_sc(degree_emb, idx).reshape(b, n, h)
    degt = jnp.transpose(deg, (0, 2, 1))
    return _materialize_bias(spd_dense.astype(jnp.int32), spd_emb, deg, degt)


# speedup vs baseline: 124.3159x; 1.0445x over previous
import functools

import jax
import jax.numpy as jnp
from jax import lax
from jax.experimental import pallas as pl
from jax.experimental.pallas import tpu as pltpu
from jax.experimental.pallas import tpu_sc as plsc

_SPD_MAX = 4
_IDX_CHUNK = 128


def _deg_gather_sc(table, idx_flat):
    bn = idx_flat.shape[0]
    h = table.shape[1]
    info = plsc.get_sparse_core_info()
    nw = info.num_cores * info.num_subcores
    per = bn // nw
    nch = per // _IDX_CHUNK
    mesh = plsc.VectorSubcoreMesh(core_axis_name="c", subcore_axis_name="s")

    @functools.partial(
        pl.kernel,
        mesh=mesh,
        compiler_params=pltpu.CompilerParams(use_tc_tiling_on_sc=False),
        out_type=jax.ShapeDtypeStruct((bn, h), jnp.float32),
        scratch_types=[
            pltpu.VMEM((per,), jnp.int32),
            pltpu.VMEM((per, h), jnp.float32),
            pltpu.SemaphoreType.DMA,
        ],
    )
    def k(table_hbm, idx_hbm, out_hbm, idx_v, rows_v, sem):
        wid = lax.axis_index("s") * info.num_cores + lax.axis_index("c")
        base = wid * per
        pltpu.sync_copy(idx_hbm.at[pl.ds(base, per)], idx_v)
        copies = [
            pltpu.async_copy(
                table_hbm.at[idx_v.at[pl.ds(j * _IDX_CHUNK, _IDX_CHUNK)]],
                rows_v.at[pl.ds(j * _IDX_CHUNK, _IDX_CHUNK)],
                sem,
            )
            for j in range(nch)
        ]
        for cp in copies:
            cp.wait()
        pltpu.sync_copy(rows_v, out_hbm.at[pl.ds(base, per)])

    return k(table, idx_flat)


def _bias_body(emb_ref, spd_ref, deg_ref, degt_ref, out_ref):
    s = jnp.minimum(spd_ref[0], _SPD_MAX)
    masks = [s == kk for kk in range(1, _SPD_MAX + 1)]
    deg = deg_ref[0]
    degt = degt_ref[0]
    n_heads = degt.shape[0]
    for hh in range(n_heads):
        x = jnp.where(masks[0], emb_ref[1, hh], emb_ref[0, hh])
        for kk in range(2, _SPD_MAX + 1):
            x = jnp.where(masks[kk - 1], emb_ref[kk, hh], x)
        out_ref[0, hh] = x + (deg[:, hh : hh + 1] + degt[hh : hh + 1, :])


def _materialize_bias(spd_dense, spd_emb, deg, degt, *, interpret=False):
    b, n, _ = spd_dense.shape
    h = spd_emb.shape[1]
    ti = 256
    grid = (b, n // ti)
    return pl.pallas_call(
        _bias_body,
        grid=grid,
        in_specs=[
            pl.BlockSpec(memory_space=pltpu.SMEM),
            pl.BlockSpec((1, ti, n), lambda bb, ii: (bb, ii, 0)),
            pl.BlockSpec((1, ti, h), lambda bb, ii: (bb, ii, 0)),
            pl.BlockSpec((1, h, n), lambda bb, ii: (bb, 0, 0)),
        ],
        out_specs=pl.BlockSpec((1, h, ti, n), lambda bb, ii: (bb, 0, ii, 0)),
        out_shape=jax.ShapeDtypeStruct((b, h, n, n), jnp.float32),
        compiler_params=pltpu.CompilerParams(
            dimension_semantics=("parallel", "parallel"),
        ),
        interpret=interpret,
    )(spd_emb, spd_dense, deg, degt)


def kernel(spd_dense, degree_dense, spd_emb, degree_emb):
    b, n, _ = spd_dense.shape
    h = spd_emb.shape[1]
    idx = degree_dense.astype(jnp.int32).reshape(b * n)
    deg = _deg_gather_sc(degree_emb, idx).reshape(b, n, h)
    degt = jnp.transpose(deg, (0, 2, 1))
    return _materialize_bias(spd_dense.astype(jnp.int32), spd_emb, deg, degt)
